# baseline (speedup 1.0000x reference)
import jax
import jax.numpy as jnp
from jax.experimental import pallas as pl

N0, N1, N2, C = 10000, 20000, 10000, 512


def _mm_body(x_ref, w_ref, o_ref):
    o_ref[...] = jnp.dot(x_ref[...], w_ref[...],
                         preferred_element_type=jnp.float32)


def _mm(x, w, bm=400):
    n, c = x.shape
    k = w.shape[1]
    assert n % bm == 0
    return pl.pallas_call(
        _mm_body,
        grid=(n // bm,),
        in_specs=[
            pl.BlockSpec((bm, c), lambda i: (i, 0)),
            pl.BlockSpec((c, k), lambda i: (0, 0)),
        ],
        out_specs=pl.BlockSpec((bm, k), lambda i: (i, 0)),
        out_shape=jax.ShapeDtypeStruct((n, k), jnp.float32),
    )(x, w)


def _spmm(rows, cols, vals, x, n_rows):
    return jnp.zeros((n_rows, x.shape[1]), x.dtype).at[rows].add(
        vals[:, None] * x[cols])


def kernel(x0, x1, x2, adj0_idx, adj0_val, adj1_idx, adj1_val, adj2_idx,
           adj2_val, inc1_rows, inc1_cols, inc1_val, inc2_rows, inc2_cols,
           inc2_val, W_same_0, W_same_1, W_same_2, W_l2h_1, W_l2h_2,
           W_h2l_0, W_h2l_1):
    ya = _mm(x0, jnp.concatenate([W_same_0, W_l2h_1], axis=1))
    yb = _mm(x1, jnp.concatenate([W_same_1, W_h2l_0, W_l2h_2], axis=1))
    yc = _mm(x2, jnp.concatenate([W_same_2, W_h2l_1], axis=1))

    y_same0, y_l2h1 = ya[:, :C], ya[:, C:]
    y_same1, y_h2l0, y_l2h2 = yb[:, :C], yb[:, C:2 * C], yb[:, 2 * C:]
    y_same2, y_h2l1 = yc[:, :C], yc[:, C:]

    m0 = _spmm(adj0_idx[0], adj0_idx[1], adj0_val, y_same0, N0)
    m0 = m0 + _spmm(inc1_rows, inc1_cols, inc1_val, y_h2l0, N0)
    m1 = _spmm(adj1_idx[0], adj1_idx[1], adj1_val, y_same1, N1)
    m1 = m1 + _spmm(inc2_rows, inc2_cols, inc2_val, y_h2l1, N1)
    m1 = m1 + _spmm(inc1_cols, inc1_rows, inc1_val, y_l2h1, N1)
    m2 = _spmm(adj2_idx[0], adj2_idx[1], adj2_val, y_same2, N2)
    m2 = m2 + _spmm(inc2_cols, inc2_rows, inc2_val, y_l2h2, N2)
    return (jax.nn.sigmoid(m0), jax.nn.sigmoid(m1), jax.nn.sigmoid(m2))
